# Initial kernel scaffold; baseline (speedup 1.0000x reference)
import functools

import jax
import jax.numpy as jnp
from jax import lax
from jax.experimental import pallas as pl
from jax.experimental.pallas import tpu as pltpu
from jax.experimental.pallas import tpu_sc as plsc

NC = 2
NS = 16
NT = NC * NS
LANES = 16


def _mesh():
    return plsc.VectorSubcoreMesh(core_axis_name="c", subcore_axis_name="s")


def _ceil_to(v, m):
    return (v + m - 1) // m * m


def _make_deg_kernel(n_pad, kc):
    rpt = n_pad // NS

    @functools.partial(
        pl.kernel,
        out_type=jax.ShapeDtypeStruct((NC, n_pad, LANES), jnp.float32),
        mesh=_mesh(),
        scratch_types=[
            pltpu.VMEM((kc, 128), jnp.int32),
            pltpu.VMEM((128, LANES), jnp.float32),
            pltpu.VMEM((rpt, LANES), jnp.float32),
            pltpu.VMEM_SHARED((n_pad, LANES), jnp.float32),
        ],
    )
    def deg_k(col_hbm, out_hbm, colv, onesv, zv, acc_sh):
        cid = lax.axis_index("c")
        sid = lax.axis_index("s")
        wid = cid * NS + sid

        def fill(j, _):
            onesv[j] = jnp.ones((LANES,), jnp.float32)
            return 0
        lax.fori_loop(0, 128, fill, 0)

        def zfill(j, _):
            zv[j] = jnp.zeros((LANES,), jnp.float32)
            return 0
        lax.fori_loop(0, rpt, zfill, 0)
        pltpu.sync_copy(zv, acc_sh.at[pl.ds(sid * rpt, rpt)])
        plsc.subcore_barrier()

        pltpu.sync_copy(col_hbm.at[wid], colv)

        def body(j, _):
            pltpu.sync_copy(onesv, acc_sh.at[colv.at[j]], add=True)
            return 0
        lax.fori_loop(0, kc, body, 0)
        plsc.subcore_barrier()

        pltpu.sync_copy(acc_sh.at[pl.ds(sid * rpt, rpt)],
                        out_hbm.at[cid, pl.ds(sid * rpt, rpt)])

    return deg_k


def _make_scat_kernel(n_pad, kc, d):
    rpt = n_pad // NS

    @functools.partial(
        pl.kernel,
        out_type=jax.ShapeDtypeStruct((NC, n_pad, d), jnp.float32),
        mesh=_mesh(),
        scratch_types=[
            pltpu.VMEM((kc, 128), jnp.int32),
            pltpu.VMEM((kc, 128), jnp.int32),
            pltpu.VMEM((128, d), jnp.float32),
            pltpu.VMEM((128, d), jnp.float32),
            pltpu.VMEM_SHARED((n_pad, d), jnp.float32),
            pltpu.SemaphoreType.DMA,
            pltpu.SemaphoreType.DMA,
        ],
    )
    def scat_k(u_hbm, row_hbm, col_hbm, out_hbm, rowv, colv, buf0, buf1,
               acc_sh, sem0, sem1):
        cid = lax.axis_index("c")
        sid = lax.axis_index("s")
        wid = cid * NS + sid

        def zfill(j, _):
            for k in range(d // LANES):
                buf0[j, pl.ds(k * LANES, LANES)] = jnp.zeros((LANES,),
                                                             jnp.float32)
            return 0
        lax.fori_loop(0, 128, zfill, 0)
        nfull, rem = rpt // 128, rpt % 128
        for t in range(nfull):
            pltpu.sync_copy(buf0, acc_sh.at[pl.ds(sid * rpt + t * 128, 128)])
        if rem:
            pltpu.sync_copy(buf0.at[pl.ds(0, rem)],
                            acc_sh.at[pl.ds(sid * rpt + nfull * 128, rem)])
        plsc.subcore_barrier()

        pltpu.sync_copy(row_hbm.at[wid], rowv)
        pltpu.sync_copy(col_hbm.at[wid], colv)

        pltpu.async_copy(u_hbm.at[rowv.at[0]], buf0, sem0)

        def body(i, _):
            j0 = 2 * i
            pltpu.async_copy(u_hbm.at[rowv.at[j0 + 1]], buf1, sem1)
            pltpu.make_async_copy(u_hbm.at[rowv.at[j0]], buf0, sem0).wait()
            pltpu.sync_copy(buf0, acc_sh.at[colv.at[j0]], add=True)
            pltpu.async_copy(u_hbm.at[rowv.at[j0 + 2]], buf0, sem0)
            pltpu.make_async_copy(u_hbm.at[rowv.at[j0 + 1]], buf1, sem1).wait()
            pltpu.sync_copy(buf1, acc_sh.at[colv.at[j0 + 1]], add=True)
            return 0

        lax.fori_loop(0, (kc - 1) // 2, body, 0)
        pltpu.make_async_copy(u_hbm.at[rowv.at[kc - 1]], buf0, sem0).wait()
        pltpu.sync_copy(buf0, acc_sh.at[colv.at[kc - 1]], add=True)
        plsc.subcore_barrier()

        pltpu.sync_copy(acc_sh.at[pl.ds(sid * rpt, rpt)],
                        out_hbm.at[cid, pl.ds(sid * rpt, rpt)])

    return scat_k


def _make_decode_kernel(kc2, d):
    @functools.partial(
        pl.kernel,
        out_type=jax.ShapeDtypeStruct((NT, kc2, 128), jnp.float32),
        mesh=_mesh(),
        scratch_types=[
            pltpu.VMEM((kc2, 128), jnp.int32),
            pltpu.VMEM((kc2, 128), jnp.int32),
            pltpu.VMEM((128, d), jnp.float32),
            pltpu.VMEM((128, d), jnp.float32),
            pltpu.VMEM((128,), jnp.float32),
            pltpu.SemaphoreType.DMA,
            pltpu.SemaphoreType.DMA,
        ],
    )
    def dec_k(z_hbm, src_hbm, dst_hbm, out_hbm, srcv, dstv, gs, gt, lg,
              sem0, sem1):
        cid = lax.axis_index("c")
        sid = lax.axis_index("s")
        wid = cid * NS + sid
        pltpu.sync_copy(src_hbm.at[wid], srcv)
        pltpu.sync_copy(dst_hbm.at[wid], dstv)

        def body(j, _):
            pltpu.async_copy(z_hbm.at[srcv.at[j]], gs, sem0)
            pltpu.async_copy(z_hbm.at[dstv.at[j]], gt, sem1)
            pltpu.make_async_copy(z_hbm.at[srcv.at[j]], gs, sem0).wait()
            pltpu.make_async_copy(z_hbm.at[dstv.at[j]], gt, sem1).wait()

            def edge(e, _):
                p = gs[e, pl.ds(0, LANES)] * gt[e, pl.ds(0, LANES)]
                for k in range(1, d // LANES):
                    p = p + (gs[e, pl.ds(k * LANES, LANES)] *
                             gt[e, pl.ds(k * LANES, LANES)])
                lg[e] = jnp.sum(p)
                return 0
            lax.fori_loop(0, 128, edge, 0)
            pltpu.sync_copy(lg, out_hbm.at[wid, j])
            return 0
        lax.fori_loop(0, kc2, body, 0)

    return dec_k


def _dinv_from_parts(degp_blk):
    deg = 1.0 + degp_blk[0, :, 0] + degp_blk[1, :, 0]
    return lax.rsqrt(deg)


def _dense1_body(x_ref, w_ref, degp_ref, u_ref):
    dinv = _dinv_from_parts(degp_ref[...])
    xw = jnp.dot(x_ref[...], w_ref[...], preferred_element_type=jnp.float32)
    u_ref[...] = xw * dinv[:, None]


def _dense2_body(p_ref, u1_ref, degp_ref, w_ref, b_ref, u2_ref):
    dinv = _dinv_from_parts(degp_ref[...])
    z1 = dinv[:, None] * (p_ref[0] + p_ref[1] + u1_ref[...]) + b_ref[...]
    z1 = jnp.maximum(z1, 0.0)
    u2_ref[...] = jnp.dot(z1, w_ref[...],
                          preferred_element_type=jnp.float32) * dinv[:, None]


def _combine_body(p_ref, u2_ref, degp_ref, b_ref, z_ref):
    dinv = _dinv_from_parts(degp_ref[...])
    z_ref[...] = dinv[:, None] * (p_ref[0] + p_ref[1] + u2_ref[...]) + b_ref[...]


def kernel(x, pos_edge_index, neg_edge_index, W1, b1, W2, b2):
    N, D = x.shape
    H = W1.shape[1]
    E = pos_edge_index.shape[1]
    E2 = 2 * E

    n_pad = _ceil_to(N, 128)
    ept = _ceil_to(-(-E // NT), 128)
    kc = ept // 128
    e_pad = NT * ept
    ept2 = _ceil_to(-(-E2 // NT), 128)
    kc2 = ept2 // 128
    e2_pad = NT * ept2

    f32 = jnp.float32
    i32 = jnp.int32

    xp = jnp.zeros((n_pad, D), f32).at[:N].set(x)
    row, col = pos_edge_index[0], pos_edge_index[1]
    rowp = jnp.concatenate(
        [row, jnp.zeros((e_pad - E,), i32)]).reshape(NT, kc, 128)
    colp = jnp.concatenate(
        [col, jnp.full((e_pad - E,), n_pad - 1, i32)]).reshape(NT, kc, 128)

    edge_index = jnp.concatenate([pos_edge_index, neg_edge_index], axis=-1)
    srcp = jnp.concatenate(
        [edge_index[0], jnp.zeros((e2_pad - E2,), i32)]).reshape(NT, kc2, 128)
    dstp = jnp.concatenate(
        [edge_index[1], jnp.zeros((e2_pad - E2,), i32)]).reshape(NT, kc2, 128)

    degp = _make_deg_kernel(n_pad, kc)(colp)

    rb = n_pad // 8
    grid = (n_pad // rb,)
    degp_spec = pl.BlockSpec((NC, rb, LANES), lambda i: (0, i, 0))
    row_spec = pl.BlockSpec((rb, D), lambda i: (i, 0))
    w_spec = pl.BlockSpec((D, H), lambda i: (0, 0))
    b_spec = pl.BlockSpec((1, H), lambda i: (0, 0))
    part_spec = pl.BlockSpec((NC, rb, H), lambda i: (0, i, 0))

    u1 = pl.pallas_call(
        _dense1_body,
        grid=grid,
        in_specs=[row_spec, w_spec, degp_spec],
        out_specs=row_spec,
        out_shape=jax.ShapeDtypeStruct((n_pad, H), f32),
    )(xp, W1, degp)

    scat = _make_scat_kernel(n_pad, kc, H)
    p1 = scat(u1, rowp, colp)

    u2 = pl.pallas_call(
        _dense2_body,
        grid=grid,
        in_specs=[part_spec, row_spec, degp_spec, w_spec, b_spec],
        out_specs=row_spec,
        out_shape=jax.ShapeDtypeStruct((n_pad, H), f32),
    )(p1, u1, degp, W2, b1.reshape(1, H))

    p2 = scat(u2, rowp, colp)

    z = pl.pallas_call(
        _combine_body,
        grid=grid,
        in_specs=[part_spec, row_spec, degp_spec, b_spec],
        out_specs=row_spec,
        out_shape=jax.ShapeDtypeStruct((n_pad, H), f32),
    )(p2, u2, degp, b2.reshape(1, H))

    logits = _make_decode_kernel(kc2, H)(z, srcp, dstp)
    logits = logits.reshape(-1)[:E2]
    return logits, edge_index


# speedup vs baseline: 8.4815x; 8.4815x over previous
import functools

import jax
import jax.numpy as jnp
from jax import lax
from jax.experimental import pallas as pl
from jax.experimental.pallas import tpu as pltpu
from jax.experimental.pallas import tpu_sc as plsc

NC = 2
NS = 16
NT = NC * NS
LANES = 16


def _mesh():
    return plsc.VectorSubcoreMesh(core_axis_name="c", subcore_axis_name="s")


def _ceil_to(v, m):
    return (v + m - 1) // m * m


def _make_deg_kernel(n_pad, kc):
    rpt = n_pad // NS

    @functools.partial(
        pl.kernel,
        out_type=jax.ShapeDtypeStruct((NC * n_pad, LANES), jnp.float32),
        mesh=_mesh(),
        compiler_params=pltpu.CompilerParams(needs_layout_passes=False),
        scratch_types=[
            pltpu.VMEM((kc, 128), jnp.int32),
            pltpu.VMEM((128, LANES), jnp.float32),
            pltpu.VMEM((128, LANES), jnp.float32),
            pltpu.VMEM((128,), jnp.int32),
            pltpu.VMEM_SHARED((n_pad, LANES), jnp.float32),
        ],
    )
    def deg_k(col_hbm, out_hbm, colv, onesv, zv, idxv, acc_sh):
        cid = lax.axis_index("c")
        sid = lax.axis_index("s")
        wid = cid * NS + sid
        lane = lax.broadcasted_iota(jnp.int32, (LANES,), 0)

        def fill(j, _):
            onesv[j] = jnp.ones((LANES,), jnp.float32)
            zv[j] = jnp.zeros((LANES,), jnp.float32)
            return 0
        lax.fori_loop(0, 128, fill, 0)

        def set_idx(base):
            for k in range(128 // LANES):
                idxv[pl.ds(k * LANES, LANES)] = base + k * LANES + lane

        for t in range(rpt // 128):
            set_idx(sid * rpt + t * 128)
            pltpu.sync_copy(zv, acc_sh.at[idxv])
        plsc.subcore_barrier()

        pltpu.sync_copy(col_hbm.at[wid], colv)

        def body(j, _):
            pltpu.sync_copy(onesv, acc_sh.at[colv.at[j]], add=True)
            return 0
        lax.fori_loop(0, kc, body, 0)
        plsc.subcore_barrier()

        for t in range(rpt // 128):
            set_idx(sid * rpt + t * 128)
            pltpu.sync_copy(acc_sh.at[idxv], zv)
            pltpu.sync_copy(
                zv, out_hbm.at[pl.ds(cid * n_pad + sid * rpt + t * 128, 128)])

    return deg_k


def _make_scat_kernel(n_pad, kc, d):
    rpt = n_pad // NS

    @functools.partial(
        pl.kernel,
        out_type=jax.ShapeDtypeStruct((NC * n_pad, d), jnp.float32),
        mesh=_mesh(),
        compiler_params=pltpu.CompilerParams(needs_layout_passes=False),
        scratch_types=[
            pltpu.VMEM((kc, 128), jnp.int32),
            pltpu.VMEM((128,), jnp.int32),
            pltpu.VMEM((128,), jnp.int32),
            pltpu.VMEM((128,), jnp.int32),
            pltpu.VMEM((128,), jnp.int32),
            pltpu.VMEM((128,), jnp.int32),
            pltpu.VMEM((128, d), jnp.float32),
            pltpu.VMEM((128, d), jnp.float32),
            pltpu.VMEM_SHARED((n_pad, d), jnp.float32),
            pltpu.SemaphoreType.DMA,
            pltpu.SemaphoreType.DMA,
        ],
    )
    def scat_k(u_hbm, rc_hbm, out_hbm, pk, ridx0, cidx0, ridx1, cidx1,
               idxv, buf0, buf1, acc_sh, sem0, sem1):
        cid = lax.axis_index("c")
        sid = lax.axis_index("s")
        wid = cid * NS + sid
        lane = lax.broadcasted_iota(jnp.int32, (LANES,), 0)

        def set_idx(base):
            for k in range(128 // LANES):
                idxv[pl.ds(k * LANES, LANES)] = base + k * LANES + lane

        def zfill(j, _):
            for k in range(d // LANES):
                buf0[j, pl.ds(k * LANES, LANES)] = jnp.zeros((LANES,),
                                                             jnp.float32)
            return 0
        lax.fori_loop(0, 128, zfill, 0)
        for t in range(rpt // 128):
            set_idx(sid * rpt + t * 128)
            pltpu.sync_copy(buf0, acc_sh.at[idxv])
        plsc.subcore_barrier()

        pltpu.sync_copy(rc_hbm.at[wid], pk)

        def unpack(j, ridx, cidx):
            for k in range(128 // LANES):
                v = pk[j, pl.ds(k * LANES, LANES)]
                ridx[pl.ds(k * LANES, LANES)] = v & 0xFFFF
                cidx[pl.ds(k * LANES, LANES)] = lax.shift_right_logical(v, 16)

        unpack(0, ridx0, cidx0)
        pltpu.async_copy(u_hbm.at[ridx0], buf0, sem0)

        def body(i, _):
            j0 = 2 * i
            unpack(j0 + 1, ridx1, cidx1)
            pltpu.make_async_copy(u_hbm.at[ridx0], buf0, sem0).wait()
            pltpu.async_copy(u_hbm.at[ridx1], buf1, sem1)
            pltpu.sync_copy(buf0, acc_sh.at[cidx0], add=True)
            unpack(j0 + 2, ridx0, cidx0)
            pltpu.make_async_copy(u_hbm.at[ridx1], buf1, sem1).wait()
            pltpu.async_copy(u_hbm.at[ridx0], buf0, sem0)
            pltpu.sync_copy(buf1, acc_sh.at[cidx1], add=True)
            return 0

        lax.fori_loop(0, (kc - 1) // 2, body, 0)
        pltpu.make_async_copy(u_hbm.at[ridx0], buf0, sem0).wait()
        pltpu.sync_copy(buf0, acc_sh.at[cidx0], add=True)
        plsc.subcore_barrier()

        base = cid * n_pad + sid * rpt
        for t in range(rpt // 128):
            set_idx(sid * rpt + t * 128)
            pltpu.sync_copy(acc_sh.at[idxv], buf0)
            pltpu.sync_copy(buf0, out_hbm.at[pl.ds(base + t * 128, 128)])

    return scat_k


def _make_decode_kernel(kc2, d):
    @functools.partial(
        pl.kernel,
        out_type=jax.ShapeDtypeStruct((NT * kc2 * 128,), jnp.float32),
        mesh=_mesh(),
        compiler_params=pltpu.CompilerParams(needs_layout_passes=False),
        scratch_types=[
            pltpu.VMEM((kc2, 128), jnp.int32),
            pltpu.VMEM((kc2, 128), jnp.int32),
            pltpu.VMEM((128, d), jnp.float32),
            pltpu.VMEM((128, d), jnp.float32),
            pltpu.VMEM((LANES, LANES), jnp.float32),
            pltpu.VMEM((128,), jnp.float32),
            pltpu.SemaphoreType.DMA,
            pltpu.SemaphoreType.DMA,
        ],
    )
    def dec_k(z_hbm, src_hbm, dst_hbm, out_hbm, srcv, dstv, gs, gt, tbuf, lg,
              sem0, sem1):
        cid = lax.axis_index("c")
        sid = lax.axis_index("s")
        wid = cid * NS + sid
        pltpu.sync_copy(src_hbm.at[wid], srcv)
        pltpu.sync_copy(dst_hbm.at[wid], dstv)

        def body(j, _):
            pltpu.async_copy(z_hbm.at[srcv.at[j]], gs, sem0)
            pltpu.async_copy(z_hbm.at[dstv.at[j]], gt, sem1)
            pltpu.make_async_copy(z_hbm.at[srcv.at[j]], gs, sem0).wait()
            pltpu.make_async_copy(z_hbm.at[dstv.at[j]], gt, sem1).wait()

            lane = lax.broadcasted_iota(jnp.int32, (LANES,), 0)

            def grp(g, _):
                for t in range(LANES):
                    e = g * LANES + t
                    p = gs[e, pl.ds(0, LANES)] * gt[e, pl.ds(0, LANES)]
                    for k in range(1, d // LANES):
                        p = p + (gs[e, pl.ds(k * LANES, LANES)] *
                                 gt[e, pl.ds(k * LANES, LANES)])
                    plsc.store_scatter(
                        tbuf, [lane, jnp.full((LANES,), t, jnp.int32)], p)
                s = tbuf[0]
                for l in range(1, LANES):
                    s = s + tbuf[l]
                lg[pl.ds(g * LANES, LANES)] = s
                return 0
            lax.fori_loop(0, 128 // LANES, grp, 0)
            pltpu.sync_copy(lg, out_hbm.at[pl.ds((wid * kc2 + j) * 128, 128)])
            return 0
        lax.fori_loop(0, kc2, body, 0)

    return dec_k


def _dinv_from_parts(degp_blk):
    deg = 1.0 + degp_blk[0, :, 0] + degp_blk[1, :, 0]
    return lax.rsqrt(deg)


def _dense1_body(x_ref, w_ref, degp_ref, u_ref):
    dinv = _dinv_from_parts(degp_ref[...])
    xw = jnp.dot(x_ref[...], w_ref[...], preferred_element_type=jnp.float32)
    u_ref[...] = xw * dinv[:, None]


def _dense2_body(p_ref, u1_ref, degp_ref, w_ref, b_ref, u2_ref):
    dinv = _dinv_from_parts(degp_ref[...])
    z1 = dinv[:, None] * (p_ref[0] + p_ref[1] + u1_ref[...]) + b_ref[...]
    z1 = jnp.maximum(z1, 0.0)
    u2_ref[...] = jnp.dot(z1, w_ref[...],
                          preferred_element_type=jnp.float32) * dinv[:, None]


def _combine_body(p_ref, u2_ref, degp_ref, b_ref, z_ref):
    dinv = _dinv_from_parts(degp_ref[...])
    z_ref[...] = dinv[:, None] * (p_ref[0] + p_ref[1] + u2_ref[...]) + b_ref[...]


def kernel(x, pos_edge_index, neg_edge_index, W1, b1, W2, b2):
    N, D = x.shape
    H = W1.shape[1]
    E = pos_edge_index.shape[1]
    E2 = 2 * E

    n_pad = _ceil_to(N, 2048)
    ept = _ceil_to(-(-E // NT), 128)
    kc = ept // 128
    e_pad = NT * ept
    ept2 = _ceil_to(-(-E2 // NT), 128)
    kc2 = ept2 // 128
    e2_pad = NT * ept2

    f32 = jnp.float32
    i32 = jnp.int32

    assert kc % 2 == 1 and n_pad < (1 << 16)
    xp = jnp.zeros((n_pad, D), f32).at[:N].set(x)
    row, col = pos_edge_index[0], pos_edge_index[1]
    rowp = jnp.concatenate(
        [row, jnp.zeros((e_pad - E,), i32)]).reshape(NT, kc, 128)
    colp = jnp.concatenate(
        [col, jnp.full((e_pad - E,), n_pad - 1, i32)]).reshape(NT, kc, 128)
    rcp = rowp | (colp << 16)

    edge_index = jnp.concatenate([pos_edge_index, neg_edge_index], axis=-1)
    srcp = jnp.concatenate(
        [edge_index[0], jnp.zeros((e2_pad - E2,), i32)]).reshape(NT, kc2, 128)
    dstp = jnp.concatenate(
        [edge_index[1], jnp.zeros((e2_pad - E2,), i32)]).reshape(NT, kc2, 128)

    degp = _make_deg_kernel(n_pad, kc)(colp).reshape(NC, n_pad, LANES)

    rb = n_pad // 8
    grid = (n_pad // rb,)
    degp_spec = pl.BlockSpec((NC, rb, LANES), lambda i: (0, i, 0))
    row_spec = pl.BlockSpec((rb, D), lambda i: (i, 0))
    w_spec = pl.BlockSpec((D, H), lambda i: (0, 0))
    b_spec = pl.BlockSpec((1, H), lambda i: (0, 0))
    part_spec = pl.BlockSpec((NC, rb, H), lambda i: (0, i, 0))

    u1 = pl.pallas_call(
        _dense1_body,
        grid=grid,
        in_specs=[row_spec, w_spec, degp_spec],
        out_specs=row_spec,
        out_shape=jax.ShapeDtypeStruct((n_pad, H), f32),
    )(xp, W1, degp)

    _DBG_SC_SCAT = True
    _DBG_SC_DEC = True
    if _DBG_SC_SCAT:
        scat = _make_scat_kernel(n_pad, kc, H)
        scat_fn = lambda u: scat(u, rcp).reshape(NC, n_pad, H)
    else:
        def scat_fn(u):
            agg = jnp.zeros((n_pad, H), f32).at[col].add(u[row])
            return jnp.stack([agg, jnp.zeros((n_pad, H), f32)])
    p1 = scat_fn(u1)

    u2 = pl.pallas_call(
        _dense2_body,
        grid=grid,
        in_specs=[part_spec, row_spec, degp_spec, w_spec, b_spec],
        out_specs=row_spec,
        out_shape=jax.ShapeDtypeStruct((n_pad, H), f32),
    )(p1, u1, degp, W2, b1.reshape(1, H))

    p2 = scat_fn(u2)

    z = pl.pallas_call(
        _combine_body,
        grid=grid,
        in_specs=[part_spec, row_spec, degp_spec, b_spec],
        out_specs=row_spec,
        out_shape=jax.ShapeDtypeStruct((n_pad, H), f32),
    )(p2, u2, degp, b2.reshape(1, H))

    if _DBG_SC_DEC:
        logits = _make_decode_kernel(kc2, H)(z, srcp, dstp)
        logits = logits.reshape(-1)[:E2]
    else:
        logits = (z[edge_index[0]] * z[edge_index[1]]).sum(axis=-1)
    return logits, edge_index


# speedup vs baseline: 10.3502x; 1.2203x over previous
import functools

import jax
import jax.numpy as jnp
from jax import lax
from jax.experimental import pallas as pl
from jax.experimental.pallas import tpu as pltpu
from jax.experimental.pallas import tpu_sc as plsc

NC = 2
NS = 16
NT = NC * NS
LANES = 16


def _mesh():
    return plsc.VectorSubcoreMesh(core_axis_name="c", subcore_axis_name="s")


def _ceil_to(v, m):
    return (v + m - 1) // m * m


def _make_deg_kernel(n_pad, kc):
    rpt = n_pad // NS

    @functools.partial(
        pl.kernel,
        out_type=jax.ShapeDtypeStruct((NC * n_pad, LANES), jnp.float32),
        mesh=_mesh(),
        compiler_params=pltpu.CompilerParams(needs_layout_passes=False),
        scratch_types=[
            pltpu.VMEM((kc, 128), jnp.int32),
            pltpu.VMEM((128, LANES), jnp.float32),
            pltpu.VMEM((128, LANES), jnp.float32),
            pltpu.VMEM((128,), jnp.int32),
            pltpu.VMEM_SHARED((n_pad, LANES), jnp.float32),
        ],
    )
    def deg_k(col_hbm, out_hbm, colv, onesv, zv, idxv, acc_sh):
        cid = lax.axis_index("c")
        sid = lax.axis_index("s")
        wid = cid * NS + sid
        lane = lax.broadcasted_iota(jnp.int32, (LANES,), 0)

        def fill(j, _):
            onesv[j] = jnp.ones((LANES,), jnp.float32)
            zv[j] = jnp.zeros((LANES,), jnp.float32)
            return 0
        lax.fori_loop(0, 128, fill, 0)

        def set_idx(base):
            for k in range(128 // LANES):
                idxv[pl.ds(k * LANES, LANES)] = base + k * LANES + lane

        for t in range(rpt // 128):
            set_idx(sid * rpt + t * 128)
            pltpu.sync_copy(zv, acc_sh.at[idxv])
        plsc.subcore_barrier()

        pltpu.sync_copy(col_hbm.at[wid], colv)

        def body(j, _):
            pltpu.sync_copy(onesv, acc_sh.at[colv.at[j]], add=True)
            return 0
        lax.fori_loop(0, kc, body, 0)
        plsc.subcore_barrier()

        for t in range(rpt // 128):
            set_idx(sid * rpt + t * 128)
            pltpu.sync_copy(acc_sh.at[idxv], zv)
            pltpu.sync_copy(
                zv, out_hbm.at[pl.ds(cid * n_pad + sid * rpt + t * 128, 128)])

    return deg_k


def _make_scat_kernel(n_pad, kc, d):
    rpt = n_pad // NS

    @functools.partial(
        pl.kernel,
        out_type=jax.ShapeDtypeStruct((NC * n_pad, d), jnp.float32),
        mesh=_mesh(),
        compiler_params=pltpu.CompilerParams(needs_layout_passes=False),
        scratch_types=[
            pltpu.VMEM((kc, 128), jnp.int32),
            pltpu.VMEM((128,), jnp.int32),
            pltpu.VMEM((128,), jnp.int32),
            pltpu.VMEM((128,), jnp.int32),
            pltpu.VMEM((128,), jnp.int32),
            pltpu.VMEM((128,), jnp.int32),
            pltpu.VMEM((128, d), jnp.float32),
            pltpu.VMEM((128, d), jnp.float32),
            pltpu.VMEM_SHARED((n_pad, d), jnp.float32),
            pltpu.SemaphoreType.DMA,
            pltpu.SemaphoreType.DMA,
            pltpu.SemaphoreType.DMA,
            pltpu.SemaphoreType.DMA,
        ],
    )
    def scat_k(u_hbm, rc_hbm, out_hbm, pk, ridx0, cidx0, ridx1, cidx1,
               idxv, buf0, buf1, acc_sh, semg0, semg1, sems0, sems1):
        cid = lax.axis_index("c")
        sid = lax.axis_index("s")
        wid = cid * NS + sid
        lane = lax.broadcasted_iota(jnp.int32, (LANES,), 0)

        def set_idx(base):
            for k in range(128 // LANES):
                idxv[pl.ds(k * LANES, LANES)] = base + k * LANES + lane

        def zfill(j, _):
            for k in range(d // LANES):
                buf0[j, pl.ds(k * LANES, LANES)] = jnp.zeros((LANES,),
                                                             jnp.float32)
            return 0
        lax.fori_loop(0, 128, zfill, 0)
        for t in range(rpt // 128):
            set_idx(sid * rpt + t * 128)
            pltpu.sync_copy(buf0, acc_sh.at[idxv])
        plsc.subcore_barrier()

        pltpu.sync_copy(rc_hbm.at[wid], pk)

        def unpack(j, ridx, cidx):
            for k in range(128 // LANES):
                v = pk[j, pl.ds(k * LANES, LANES)]
                ridx[pl.ds(k * LANES, LANES)] = v & 0xFFFF
                cidx[pl.ds(k * LANES, LANES)] = lax.shift_right_logical(v, 16)

        def wait_g(ridx, buf, semg):
            pltpu.make_async_copy(u_hbm.at[ridx], buf, semg).wait()

        def start_s(buf, cidx, sems):
            pltpu.async_copy(buf, acc_sh.at[cidx], sems, add=True)

        def wait_s(buf, cidx, sems):
            pltpu.make_async_copy(buf, acc_sh.at[cidx], sems).wait()

        unpack(0, ridx0, cidx0)
        pltpu.async_copy(u_hbm.at[ridx0], buf0, semg0)

        def body(i, _):
            j0 = 2 * i
            unpack(j0 + 1, ridx1, cidx1)
            wait_g(ridx0, buf0, semg0)
            pltpu.async_copy(u_hbm.at[ridx1], buf1, semg1)
            pltpu.sync_copy(buf0, acc_sh.at[cidx0], add=True)
            unpack(j0 + 2, ridx0, cidx0)
            wait_g(ridx1, buf1, semg1)
            pltpu.async_copy(u_hbm.at[ridx0], buf0, semg0)
            pltpu.sync_copy(buf1, acc_sh.at[cidx1], add=True)
            return 0

        lax.fori_loop(0, (kc - 1) // 2, body, 0)
        wait_g(ridx0, buf0, semg0)
        pltpu.sync_copy(buf0, acc_sh.at[cidx0], add=True)
        plsc.subcore_barrier()

        base = cid * n_pad + sid * rpt
        for t in range(rpt // 128):
            set_idx(sid * rpt + t * 128)
            pltpu.sync_copy(acc_sh.at[idxv], buf0)
            pltpu.sync_copy(buf0, out_hbm.at[pl.ds(base + t * 128, 128)])

    return scat_k


def _make_decode_kernel(kc2, d):
    @functools.partial(
        pl.kernel,
        out_type=jax.ShapeDtypeStruct((NT, kc2, 128), jnp.float32),
        mesh=_mesh(),
        compiler_params=pltpu.CompilerParams(needs_layout_passes=False),
        scratch_types=[
            pltpu.VMEM((kc2, 128), jnp.int32),
            pltpu.VMEM((kc2, 128), jnp.int32),
            pltpu.VMEM((128, d), jnp.float32),
            pltpu.VMEM((128, d), jnp.float32),
            pltpu.VMEM((128, d), jnp.float32),
            pltpu.VMEM((128, d), jnp.float32),
            pltpu.VMEM((LANES, LANES), jnp.float32),
            pltpu.VMEM((kc2, 128), jnp.float32),
            pltpu.SemaphoreType.DMA,
            pltpu.SemaphoreType.DMA,
            pltpu.SemaphoreType.DMA,
            pltpu.SemaphoreType.DMA,
        ],
    )
    def dec_k(z_hbm, src_hbm, dst_hbm, out_hbm, srcv, dstv, gs0, gt0,
              gs1, gt1, tbuf, lgbuf, sa0, sb0, sa1, sb1):
        cid = lax.axis_index("c")
        sid = lax.axis_index("s")
        wid = cid * NS + sid
        pltpu.sync_copy(src_hbm.at[wid], srcv)
        pltpu.sync_copy(dst_hbm.at[wid], dstv)
        lane = lax.broadcasted_iota(jnp.int32, (LANES,), 0)

        def start(j, gs, gt, sa, sb):
            pltpu.async_copy(z_hbm.at[srcv.at[j]], gs, sa)
            pltpu.async_copy(z_hbm.at[dstv.at[j]], gt, sb)

        def wait(j, gs, gt, sa, sb):
            pltpu.make_async_copy(z_hbm.at[srcv.at[j]], gs, sa).wait()
            pltpu.make_async_copy(z_hbm.at[dstv.at[j]], gt, sb).wait()

        def compute(j, gs, gt):
            def grp(g, _):
                for t in range(LANES):
                    e = g * LANES + t
                    p = gs[e, pl.ds(0, LANES)] * gt[e, pl.ds(0, LANES)]
                    for k in range(1, d // LANES):
                        p = p + (gs[e, pl.ds(k * LANES, LANES)] *
                                 gt[e, pl.ds(k * LANES, LANES)])
                    plsc.store_scatter(
                        tbuf, [lane, jnp.full((LANES,), t, jnp.int32)], p)
                s = tbuf[0]
                for l in range(1, LANES):
                    s = s + tbuf[l]
                lgbuf[j, pl.ds(g * LANES, LANES)] = s
                return 0
            lax.fori_loop(0, 128 // LANES, grp, 0)

        start(0, gs0, gt0, sa0, sb0)

        def body(i, _):
            j0 = 2 * i
            start(j0 + 1, gs1, gt1, sa1, sb1)
            wait(j0, gs0, gt0, sa0, sb0)
            compute(j0, gs0, gt0)
            start(j0 + 2, gs0, gt0, sa0, sb0)
            wait(j0 + 1, gs1, gt1, sa1, sb1)
            compute(j0 + 1, gs1, gt1)
            return 0

        lax.fori_loop(0, (kc2 - 1) // 2, body, 0)
        wait(kc2 - 1, gs0, gt0, sa0, sb0)
        compute(kc2 - 1, gs0, gt0)
        pltpu.sync_copy(lgbuf, out_hbm.at[wid])

    return dec_k


def _dinv_from_parts(degp_blk):
    deg = 1.0 + degp_blk[0, :, 0] + degp_blk[1, :, 0]
    return lax.rsqrt(deg)


def _dense1_body(x_ref, w_ref, degp_ref, u_ref):
    dinv = _dinv_from_parts(degp_ref[...])
    xw = jnp.dot(x_ref[...], w_ref[...], preferred_element_type=jnp.float32)
    u_ref[...] = xw * dinv[:, None]


def _dense2_body(p_ref, u1_ref, degp_ref, w_ref, b_ref, u2_ref):
    dinv = _dinv_from_parts(degp_ref[...])
    z1 = dinv[:, None] * (p_ref[0] + p_ref[1] + u1_ref[...]) + b_ref[...]
    z1 = jnp.maximum(z1, 0.0)
    u2_ref[...] = jnp.dot(z1, w_ref[...],
                          preferred_element_type=jnp.float32) * dinv[:, None]


def _combine_body(p_ref, u2_ref, degp_ref, b_ref, z_ref):
    dinv = _dinv_from_parts(degp_ref[...])
    z_ref[...] = dinv[:, None] * (p_ref[0] + p_ref[1] + u2_ref[...]) + b_ref[...]


def kernel(x, pos_edge_index, neg_edge_index, W1, b1, W2, b2):
    N, D = x.shape
    H = W1.shape[1]
    E = pos_edge_index.shape[1]
    E2 = 2 * E

    n_pad = _ceil_to(N, 2048)
    ept = _ceil_to(-(-E // NT), 128)
    kc = ept // 128
    e_pad = NT * ept
    ept2 = _ceil_to(-(-E2 // NT), 128)
    kc2 = ept2 // 128
    e2_pad = NT * ept2

    f32 = jnp.float32
    i32 = jnp.int32

    assert kc % 2 == 1 and kc2 % 2 == 1 and n_pad < (1 << 16)
    xp = jnp.zeros((n_pad, D), f32).at[:N].set(x)
    row, col = pos_edge_index[0], pos_edge_index[1]
    rowp = jnp.concatenate(
        [row, jnp.zeros((e_pad - E,), i32)]).reshape(NT, kc, 128)
    colp = jnp.concatenate(
        [col, jnp.full((e_pad - E,), n_pad - 1, i32)]).reshape(NT, kc, 128)
    rcp = rowp | (colp << 16)

    edge_index = jnp.concatenate([pos_edge_index, neg_edge_index], axis=-1)
    srcp = jnp.concatenate(
        [edge_index[0], jnp.zeros((e2_pad - E2,), i32)]).reshape(NT, kc2, 128)
    dstp = jnp.concatenate(
        [edge_index[1], jnp.zeros((e2_pad - E2,), i32)]).reshape(NT, kc2, 128)

    degp = _make_deg_kernel(n_pad, kc)(colp).reshape(NC, n_pad, LANES)

    rb = n_pad // 8
    grid = (n_pad // rb,)
    degp_spec = pl.BlockSpec((NC, rb, LANES), lambda i: (0, i, 0))
    row_spec = pl.BlockSpec((rb, D), lambda i: (i, 0))
    w_spec = pl.BlockSpec((D, H), lambda i: (0, 0))
    b_spec = pl.BlockSpec((1, H), lambda i: (0, 0))
    part_spec = pl.BlockSpec((NC, rb, H), lambda i: (0, i, 0))

    u1 = pl.pallas_call(
        _dense1_body,
        grid=grid,
        in_specs=[row_spec, w_spec, degp_spec],
        out_specs=row_spec,
        out_shape=jax.ShapeDtypeStruct((n_pad, H), f32),
    )(xp, W1, degp)

    _DBG_SC_SCAT = True
    _DBG_SC_DEC = True
    if _DBG_SC_SCAT:
        scat = _make_scat_kernel(n_pad, kc, H)
        scat_fn = lambda u: scat(u, rcp).reshape(NC, n_pad, H)
    else:
        def scat_fn(u):
            agg = jnp.zeros((n_pad, H), f32).at[col].add(u[row])
            return jnp.stack([agg, jnp.zeros((n_pad, H), f32)])
    p1 = scat_fn(u1)

    u2 = pl.pallas_call(
        _dense2_body,
        grid=grid,
        in_specs=[part_spec, row_spec, degp_spec, w_spec, b_spec],
        out_specs=row_spec,
        out_shape=jax.ShapeDtypeStruct((n_pad, H), f32),
    )(p1, u1, degp, W2, b1.reshape(1, H))

    p2 = scat_fn(u2)

    z = pl.pallas_call(
        _combine_body,
        grid=grid,
        in_specs=[part_spec, row_spec, degp_spec, b_spec],
        out_specs=row_spec,
        out_shape=jax.ShapeDtypeStruct((n_pad, H), f32),
    )(p2, u2, degp, b2.reshape(1, H))

    if _DBG_SC_DEC:
        logits = _make_decode_kernel(kc2, H)(z, srcp, dstp)
        logits = logits.reshape(-1)[:E2]
    else:
        logits = (z[edge_index[0]] * z[edge_index[1]]).sum(axis=-1)
    return logits, edge_index
